# baseline (speedup 1.0000x reference)
import functools

import jax
import jax.numpy as jnp
from jax import lax
from jax.experimental import pallas as pl
from jax.experimental.pallas import tpu as pltpu
from jax.experimental.pallas import tpu_sc as plsc

N_TASKS = 1024
N_SPLITS = 8
N_SKILLS = 64
D = N_SPLITS * N_SKILLS
B = 16384
EPS = 1e-12

_NC = 2
_NS = 16
_L = 16
_NW = _NC * _NS

_ROWS_PER_TILE = N_TASKS // _NS
_B_PER_W = B // _NW
_CH = 128
_NCH = _B_PER_W // _CH


def _body(table_hbm, ids_hbm, out_hbm, norm_hbm, tbuf, idx_v, rows_v, sem):
    cid = lax.axis_index("c")
    sid = lax.axis_index("s")

    base_row = sid * _ROWS_PER_TILE
    pltpu.sync_copy(table_hbm.at[pl.ds(base_row, _ROWS_PER_TILE)], tbuf)

    lane = lax.iota(jnp.int32, _L)
    perms = [jnp.bitwise_xor(lane, k) for k in (8, 4, 2, 1)]

    def row_body(r, carry):
        for g in range(N_SPLITS):
            col0 = g * N_SKILLS
            ss = []
            for j in range(N_SKILLS // _L):
                x = tbuf[r, pl.ds(col0 + j * _L, _L)]
                ss.append(1.0 / (1.0 + jnp.exp(-x)))
            tot = ss[0] + ss[1] + ss[2] + ss[3]
            for p in perms:
                tot = tot + tot[p]
            scale = 1.0 / (tot + EPS)
            for j in range(N_SKILLS // _L):
                tbuf[r, pl.ds(col0 + j * _L, _L)] = ss[j] * scale
        return carry

    lax.fori_loop(0, _ROWS_PER_TILE, row_body, 0)
    pltpu.sync_copy(tbuf, norm_hbm.at[cid].at[pl.ds(base_row, _ROWS_PER_TILE)])
    plsc.subcore_barrier()

    wid = sid * _NC + cid
    base = wid * _B_PER_W
    for ch in range(_NCH):
        off = base + ch * _CH
        pltpu.sync_copy(ids_hbm.at[pl.ds(off, _CH)], idx_v)
        pltpu.async_copy(norm_hbm.at[cid].at[idx_v], rows_v, sem).wait()
        pltpu.sync_copy(rows_v, out_hbm.at[pl.ds(off, _CH)])


_mesh = plsc.VectorSubcoreMesh(core_axis_name="c", subcore_axis_name="s")

_k = functools.partial(
    pl.kernel,
    mesh=_mesh,
    out_type=jax.ShapeDtypeStruct((B, D), jnp.float32),
    scratch_types=[
        pltpu.MemorySpace.HBM((_NC, N_TASKS, D), jnp.float32),
        pltpu.VMEM((_ROWS_PER_TILE, D), jnp.float32),
        pltpu.VMEM((_CH,), jnp.int32),
        pltpu.VMEM((_CH, D), jnp.float32),
        pltpu.SemaphoreType.DMA,
    ],
)(_body)


@jax.jit
def kernel(module_logits, task_ids):
    out = _k(module_logits, task_ids.astype(jnp.int32))
    return out.reshape(-1, N_SPLITS, N_SKILLS)


# speedup vs baseline: 1.0644x; 1.0644x over previous
import functools

import jax
import jax.numpy as jnp
from jax import lax
from jax.experimental import pallas as pl
from jax.experimental.pallas import tpu as pltpu
from jax.experimental.pallas import tpu_sc as plsc

N_TASKS = 1024
N_SPLITS = 8
N_SKILLS = 64
D = N_SPLITS * N_SKILLS
B = 16384
EPS = 1e-12

_NC = 2
_NS = 16
_NW = _NC * _NS

_B_PER_W = B // _NW
_CH = 64
_NCH = _B_PER_W // _CH
_NBUF = 3


def _norm_body(table_ref, out_ref):
    x = table_ref[...]
    s = 1.0 / (1.0 + jnp.exp(-x))
    for g in range(N_SPLITS):
        sl = slice(g * N_SKILLS, (g + 1) * N_SKILLS)
        grp = s[:, sl]
        tot = jnp.sum(grp, axis=1, keepdims=True) + EPS
        out_ref[:, sl] = grp / tot


_normalize = pl.pallas_call(
    _norm_body,
    out_shape=jax.ShapeDtypeStruct((N_TASKS, D), jnp.float32),
)


def _gather_body(norm_hbm, ids_hbm, out_hbm, idx_v,
                 rb0, rb1, rb2, g0, g1, g2, s0, s1, s2):
    rbufs = [rb0, rb1, rb2]
    gsems = [g0, g1, g2]
    ssems = [s0, s1, s2]

    wid = lax.axis_index("s") * _NC + lax.axis_index("c")
    base = wid * _B_PER_W
    pltpu.sync_copy(ids_hbm.at[wid], idx_v)

    def fire_gather(ch):
        b = ch % _NBUF
        return pltpu.async_copy(norm_hbm.at[idx_v.at[ch]], rbufs[b], gsems[b])

    gops = [None] * _NCH
    sops = [None] * _NCH
    for ch in range(_NBUF):
        gops[ch] = fire_gather(ch)
    for ch in range(_NCH):
        b = ch % _NBUF
        gops[ch].wait()
        if ch >= 1:
            sops[ch - 1].wait()
            nxt = ch - 1 + _NBUF
            if nxt < _NCH:
                gops[nxt] = fire_gather(nxt)
        sops[ch] = pltpu.async_copy(
            rbufs[b], out_hbm.at[pl.ds(base + ch * _CH, _CH)], ssems[b])
    sops[_NCH - 1].wait()


_mesh = plsc.VectorSubcoreMesh(core_axis_name="c", subcore_axis_name="s")

_gather = functools.partial(
    pl.kernel,
    mesh=_mesh,
    out_type=jax.ShapeDtypeStruct((B, D), jnp.float32),
    scratch_types=[
        pltpu.VMEM((_NCH, _CH), jnp.int32),
        pltpu.VMEM((_CH, D), jnp.float32),
        pltpu.VMEM((_CH, D), jnp.float32),
        pltpu.VMEM((_CH, D), jnp.float32),
        pltpu.SemaphoreType.DMA,
        pltpu.SemaphoreType.DMA,
        pltpu.SemaphoreType.DMA,
        pltpu.SemaphoreType.DMA,
        pltpu.SemaphoreType.DMA,
        pltpu.SemaphoreType.DMA,
    ],
)(_gather_body)


@jax.jit
def kernel(module_logits, task_ids):
    norm = _normalize(module_logits)
    ids = task_ids.astype(jnp.int32).reshape(_NW, _NCH, _CH)
    out = _gather(norm, ids)
    return out.reshape(-1, N_SPLITS, N_SKILLS)
